# Initial kernel scaffold; baseline (speedup 1.0000x reference)
import jax
import jax.numpy as jnp
from jax.experimental import pallas as pl


def kernel(x, Wqkv, bqkv, Wo, bo, ln1_s, ln1_b, ln2_s, ln2_b, W1, b1, W2, b2):
    raise NotImplementedError("write your pallas kernel here")


# speedup vs baseline: 1.0332x; 1.0332x over previous
import functools

import jax
import jax.numpy as jnp
from jax.experimental import pallas as pl
from jax.experimental.pallas import tpu as pltpu

B, S, D = 1, 2048, 1024
H = 16
DH = D // H
L = 2
E = 8
F = 2048
NUM_OPS = 8
CAP = (B * S // E) * 5 // 4
ECAP = E * CAP

_EPS = 1e-5


def _routing_kernel(xop_ref, flat_ref):
    xop = xop_ref[...]
    cols = jax.lax.broadcasted_iota(jnp.int32, (S, 128), 1).astype(jnp.float32)
    valid = cols < NUM_OPS
    neg = jnp.float32(-3e38)
    vals = jnp.where(valid, xop, neg)
    rowmax = jnp.max(vals, axis=1, keepdims=True)
    ismax = vals == rowmax
    opcode = jnp.min(jnp.where(ismax, cols, jnp.float32(1e9)), axis=1,
                     keepdims=True)
    onehot = jnp.where((cols == opcode) & valid, 1.0, 0.0)
    ri = jax.lax.broadcasted_iota(jnp.int32, (S, S), 0)
    ci = jax.lax.broadcasted_iota(jnp.int32, (S, S), 1)
    tril = jnp.where(ci <= ri, jnp.float32(1.0), jnp.float32(0.0))
    cum = jax.lax.dot(tril, onehot, preferred_element_type=jnp.float32)
    pos = jnp.sum(cum * onehot, axis=1, keepdims=True) - 1.0
    keep = pos < CAP
    flat = jnp.where(keep, opcode * CAP + pos, jnp.float32(ECAP))
    flat_ref[...] = flat


def _routing(xop):
    return pl.pallas_call(
        _routing_kernel,
        out_shape=jax.ShapeDtypeStruct((S, 1), jnp.float32),
    )(xop)


def _ln_kernel(h_ref, s_ref, b_ref, o_ref):
    h = h_ref[...]
    m = jnp.mean(h, axis=1, keepdims=True)
    c = h - m
    v = jnp.mean(c * c, axis=1, keepdims=True)
    o_ref[...] = c * jax.lax.rsqrt(v + _EPS) * s_ref[...] + b_ref[...]


def _layernorm(h, s, b):
    return pl.pallas_call(
        _ln_kernel,
        out_shape=jax.ShapeDtypeStruct((S, D), jnp.float32),
    )(h, s.reshape(1, D), b.reshape(1, D))


def _qkv_kernel(a_ref, wq_ref, wk_ref, wv_ref, bq_ref, bk_ref, bv_ref,
                q_ref, k_ref, v_ref):
    a = a_ref[...]
    q_ref[0] = jax.lax.dot(a, wq_ref[0], preferred_element_type=jnp.float32) + bq_ref[0]
    k_ref[0] = jax.lax.dot(a, wk_ref[0], preferred_element_type=jnp.float32) + bk_ref[0]
    v_ref[0] = jax.lax.dot(a, wv_ref[0], preferred_element_type=jnp.float32) + bv_ref[0]


def _qkv(a, wq3, wk3, wv3, bq3, bk3, bv3):
    wspec = pl.BlockSpec((1, D, DH), lambda h: (h, 0, 0))
    bspec = pl.BlockSpec((1, 1, DH), lambda h: (h, 0, 0))
    ospec = pl.BlockSpec((1, S, DH), lambda h: (h, 0, 0))
    out = jax.ShapeDtypeStruct((H, S, DH), jnp.float32)
    return pl.pallas_call(
        _qkv_kernel,
        grid=(H,),
        in_specs=[pl.BlockSpec((S, D), lambda h: (0, 0)),
                  wspec, wspec, wspec, bspec, bspec, bspec],
        out_specs=[ospec, ospec, ospec],
        out_shape=[out, out, out],
    )(a, wq3, wk3, wv3, bq3, bk3, bv3)


def _attn_kernel(q_ref, k_ref, v_ref, o_ref):
    q = q_ref[0]
    k = k_ref[0]
    v = v_ref[0]
    s = jax.lax.dot_general(q, k, (((1,), (1,)), ((), ())),
                            preferred_element_type=jnp.float32)
    s = s * jnp.float32(1.0 / (DH ** 0.5))
    ri = jax.lax.broadcasted_iota(jnp.int32, (S, S), 0)
    ci = jax.lax.broadcasted_iota(jnp.int32, (S, S), 1)
    s = jnp.where(ci <= ri, s, jnp.float32(-1e9))
    m = jnp.max(s, axis=1, keepdims=True)
    p = jnp.exp(s - m)
    denom = jnp.sum(p, axis=1, keepdims=True)
    o = jax.lax.dot(p, v, preferred_element_type=jnp.float32)
    o_ref[0] = o / denom


def _attention(q3, k3, v3):
    spec = pl.BlockSpec((1, S, DH), lambda h: (h, 0, 0))
    return pl.pallas_call(
        _attn_kernel,
        grid=(H,),
        in_specs=[spec, spec, spec],
        out_specs=spec,
        out_shape=jax.ShapeDtypeStruct((H, S, DH), jnp.float32),
    )(q3, k3, v3)


def _proj_kernel(o_ref, wo_ref, bo_ref, hin_ref, s2_ref, b2_ref,
                 hout_ref, m_ref):
    h = pl.program_id(0)

    @pl.when(h == 0)
    def _():
        hout_ref[...] = hin_ref[...] + bo_ref[...]

    hout_ref[...] += jax.lax.dot(o_ref[0], wo_ref[0],
                                 preferred_element_type=jnp.float32)

    @pl.when(h == H - 1)
    def _():
        hh = hout_ref[...]
        mu = jnp.mean(hh, axis=1, keepdims=True)
        c = hh - mu
        va = jnp.mean(c * c, axis=1, keepdims=True)
        m_ref[...] = c * jax.lax.rsqrt(va + _EPS) * s2_ref[...] + b2_ref[...]


def _proj(o3, wo3, bo, hin, s2, b2):
    full = pl.BlockSpec((S, D), lambda h: (0, 0))
    row = pl.BlockSpec((1, D), lambda h: (0, 0))
    return pl.pallas_call(
        _proj_kernel,
        grid=(H,),
        in_specs=[pl.BlockSpec((1, S, DH), lambda h: (h, 0, 0)),
                  pl.BlockSpec((1, DH, D), lambda h: (h, 0, 0)),
                  row, full, row, row],
        out_specs=[full, full],
        out_shape=[jax.ShapeDtypeStruct((S, D), jnp.float32),
                   jax.ShapeDtypeStruct((S, D), jnp.float32)],
    )(o3, wo3, bo.reshape(1, D), hin, s2.reshape(1, D), b2.reshape(1, D))


def _ffn_kernel(flat_ref, m_ref, w1_ref, b1_ref, w2_ref, b2_ref, out_ref):
    e = pl.program_id(0)
    flat = flat_ref[...]
    slot = jax.lax.broadcasted_iota(jnp.int32, (S, CAP), 1).astype(
        jnp.float32) + (jnp.float32(CAP) * e.astype(jnp.float32))
    p = jnp.where(flat == slot, 1.0, 0.0)
    ein = jax.lax.dot_general(p, m_ref[...], (((0,), (0,)), ((), ())),
                              preferred_element_type=jnp.float32)
    hid = jax.lax.dot(ein, w1_ref[0], preferred_element_type=jnp.float32)
    hid = jnp.maximum(hid + b1_ref[0], 0.0)
    out_ref[...] = jax.lax.dot(hid, w2_ref[0],
                               preferred_element_type=jnp.float32) + b2_ref[0]


def _ffn(flat, m_in, w1, b1, w2, b2):
    return pl.pallas_call(
        _ffn_kernel,
        grid=(E,),
        in_specs=[pl.BlockSpec((S, 1), lambda e: (0, 0)),
                  pl.BlockSpec((S, D), lambda e: (0, 0)),
                  pl.BlockSpec((1, D, F), lambda e: (e, 0, 0)),
                  pl.BlockSpec((1, 1, F), lambda e: (e, 0, 0)),
                  pl.BlockSpec((1, F, D), lambda e: (e, 0, 0)),
                  pl.BlockSpec((1, 1, D), lambda e: (e, 0, 0))],
        out_specs=pl.BlockSpec((CAP, D), lambda e: (e, 0)),
        out_shape=jax.ShapeDtypeStruct((ECAP, D), jnp.float32),
    )(flat, m_in, w1, b1.reshape(E, 1, F), w2, b2.reshape(E, 1, D))


_TB = 512


def _combine_kernel(flat_ref, h_ref, eout_ref, o_ref):
    flat = flat_ref[...]
    slot = jax.lax.broadcasted_iota(jnp.int32, (_TB, ECAP), 1).astype(
        jnp.float32)
    p = jnp.where(flat == slot, 1.0, 0.0)
    y = jax.lax.dot(p, eout_ref[...], preferred_element_type=jnp.float32)
    o_ref[...] = h_ref[...] + y


def _combine(flat, h, eout):
    return pl.pallas_call(
        _combine_kernel,
        grid=(S // _TB,),
        in_specs=[pl.BlockSpec((_TB, 1), lambda t: (t, 0)),
                  pl.BlockSpec((_TB, D), lambda t: (t, 0)),
                  pl.BlockSpec((ECAP, D), lambda t: (0, 0))],
        out_specs=pl.BlockSpec((_TB, D), lambda t: (t, 0)),
        out_shape=jax.ShapeDtypeStruct((S, D), jnp.float32),
    )(flat, h, eout)


def kernel(x, Wqkv, bqkv, Wo, bo, ln1_s, ln1_b, ln2_s, ln2_b, W1, b1, W2, b2):
    xs = x[0]
    flat = _routing(xs[:, :128])

    h = xs
    for l in range(L):
        a = _layernorm(h, ln1_s[l], ln1_b[l])
        wq3 = Wqkv[l, :, :D].reshape(D, H, DH).transpose(1, 0, 2)
        wk3 = Wqkv[l, :, D:2 * D].reshape(D, H, DH).transpose(1, 0, 2)
        wv3 = Wqkv[l, :, 2 * D:].reshape(D, H, DH).transpose(1, 0, 2)
        bq3 = bqkv[l, :D].reshape(H, 1, DH)
        bk3 = bqkv[l, D:2 * D].reshape(H, 1, DH)
        bv3 = bqkv[l, 2 * D:].reshape(H, 1, DH)
        q3, k3, v3 = _qkv(a, wq3, wk3, wv3, bq3, bk3, bv3)
        o3 = _attention(q3, k3, v3)
        wo3 = Wo[l].reshape(H, DH, D)
        h, m_in = _proj(o3, wo3, bo[l], h, ln2_s[l], ln2_b[l])
        eout = _ffn(flat, m_in, W1[l], b1[l], W2[l], b2[l])
        h = _combine(flat, h, eout)

    return h.reshape(B, S, D)


# speedup vs baseline: 1.0622x; 1.0281x over previous
import functools

import jax
import jax.numpy as jnp
from jax.experimental import pallas as pl
from jax.experimental.pallas import tpu as pltpu

B, S, D = 1, 2048, 1024
H = 16
DH = D // H
L = 2
E = 8
F = 2048
NUM_OPS = 8
CAP = (B * S // E) * 5 // 4
ECAP = E * CAP

_EPS = 1e-5


def _routing_kernel(xop_ref, flat_ref):
    xop = xop_ref[...]
    cols = jax.lax.broadcasted_iota(jnp.int32, (S, 128), 1).astype(jnp.float32)
    valid = cols < NUM_OPS
    neg = jnp.float32(-3e38)
    vals = jnp.where(valid, xop, neg)
    rowmax = jnp.max(vals, axis=1, keepdims=True)
    ismax = vals == rowmax
    opcode = jnp.min(jnp.where(ismax, cols, jnp.float32(1e9)), axis=1,
                     keepdims=True)
    onehot = jnp.where((cols == opcode) & valid, 1.0, 0.0)
    ri = jax.lax.broadcasted_iota(jnp.int32, (S, S), 0)
    ci = jax.lax.broadcasted_iota(jnp.int32, (S, S), 1)
    tril = jnp.where(ci <= ri, jnp.float32(1.0),
                     jnp.float32(0.0)).astype(jnp.bfloat16)
    cum = jax.lax.dot(tril, onehot.astype(jnp.bfloat16),
                      preferred_element_type=jnp.float32)
    pos = jnp.sum(cum * onehot, axis=1, keepdims=True) - 1.0
    keep = pos < CAP
    flat = jnp.where(keep, opcode * CAP + pos, jnp.float32(ECAP))
    flat_ref[...] = flat


def _routing(xop):
    return pl.pallas_call(
        _routing_kernel,
        out_shape=jax.ShapeDtypeStruct((S, 1), jnp.float32),
    )(xop)


def _ln_kernel(h_ref, s_ref, b_ref, o_ref):
    h = h_ref[...]
    m = jnp.mean(h, axis=1, keepdims=True)
    c = h - m
    v = jnp.mean(c * c, axis=1, keepdims=True)
    o_ref[...] = c * jax.lax.rsqrt(v + _EPS) * s_ref[...] + b_ref[...]


def _layernorm(h, s, b):
    return pl.pallas_call(
        _ln_kernel,
        out_shape=jax.ShapeDtypeStruct((S, D), jnp.float32),
    )(h, s.reshape(1, D), b.reshape(1, D))


def _qkv_kernel(a_ref, wq_ref, wk_ref, wv_ref, bq_ref, bk_ref, bv_ref,
                q_ref, k_ref, v_ref):
    a = a_ref[...].astype(jnp.bfloat16)
    q_ref[0] = (jax.lax.dot(a, wq_ref[0], preferred_element_type=jnp.float32)
                + bq_ref[0]).astype(jnp.bfloat16)
    k_ref[0] = (jax.lax.dot(a, wk_ref[0], preferred_element_type=jnp.float32)
                + bk_ref[0]).astype(jnp.bfloat16)
    v_ref[0] = (jax.lax.dot(a, wv_ref[0], preferred_element_type=jnp.float32)
                + bv_ref[0]).astype(jnp.bfloat16)


def _qkv(a, wq3, wk3, wv3, bq3, bk3, bv3):
    wspec = pl.BlockSpec((1, D, DH), lambda h: (h, 0, 0))
    bspec = pl.BlockSpec((1, 1, DH), lambda h: (h, 0, 0))
    ospec = pl.BlockSpec((1, S, DH), lambda h: (h, 0, 0))
    out = jax.ShapeDtypeStruct((H, S, DH), jnp.bfloat16)
    return pl.pallas_call(
        _qkv_kernel,
        grid=(H,),
        in_specs=[pl.BlockSpec((S, D), lambda h: (0, 0)),
                  wspec, wspec, wspec, bspec, bspec, bspec],
        out_specs=[ospec, ospec, ospec],
        out_shape=[out, out, out],
    )(a, wq3, wk3, wv3, bq3, bk3, bv3)


def _attn_kernel(q_ref, k_ref, v_ref, o_ref):
    q = q_ref[0]
    k = k_ref[0]
    v = v_ref[0]
    s = jax.lax.dot_general(q, k, (((1,), (1,)), ((), ())),
                            preferred_element_type=jnp.float32)
    s = s * jnp.float32(1.0 / (DH ** 0.5))
    ri = jax.lax.broadcasted_iota(jnp.int32, (S, S), 0)
    ci = jax.lax.broadcasted_iota(jnp.int32, (S, S), 1)
    s = jnp.where(ci <= ri, s, jnp.float32(-1e9))
    m = jnp.max(s, axis=1, keepdims=True)
    p = jnp.exp(s - m)
    denom = jnp.sum(p, axis=1, keepdims=True)
    o = jax.lax.dot(p.astype(jnp.bfloat16), v,
                    preferred_element_type=jnp.float32)
    o_ref[0] = (o / denom).astype(jnp.bfloat16)


def _attention(q3, k3, v3):
    spec = pl.BlockSpec((1, S, DH), lambda h: (h, 0, 0))
    return pl.pallas_call(
        _attn_kernel,
        grid=(H,),
        in_specs=[spec, spec, spec],
        out_specs=spec,
        out_shape=jax.ShapeDtypeStruct((H, S, DH), jnp.bfloat16),
    )(q3, k3, v3)


def _proj_kernel(o_ref, wo_ref, bo_ref, hin_ref, s2_ref, b2_ref,
                 hout_ref, m_ref):
    h = pl.program_id(0)

    @pl.when(h == 0)
    def _():
        hout_ref[...] = hin_ref[...] + bo_ref[...]

    hout_ref[...] += jax.lax.dot(o_ref[0], wo_ref[0],
                                 preferred_element_type=jnp.float32)

    @pl.when(h == H - 1)
    def _():
        hh = hout_ref[...]
        mu = jnp.mean(hh, axis=1, keepdims=True)
        c = hh - mu
        va = jnp.mean(c * c, axis=1, keepdims=True)
        m_ref[...] = (c * jax.lax.rsqrt(va + _EPS) * s2_ref[...]
                      + b2_ref[...]).astype(jnp.bfloat16)


def _proj(o3, wo3, bo, hin, s2, b2):
    full = pl.BlockSpec((S, D), lambda h: (0, 0))
    row = pl.BlockSpec((1, D), lambda h: (0, 0))
    return pl.pallas_call(
        _proj_kernel,
        grid=(H,),
        in_specs=[pl.BlockSpec((1, S, DH), lambda h: (h, 0, 0)),
                  pl.BlockSpec((1, DH, D), lambda h: (h, 0, 0)),
                  row, full, row, row],
        out_specs=[full, full],
        out_shape=[jax.ShapeDtypeStruct((S, D), jnp.float32),
                   jax.ShapeDtypeStruct((S, D), jnp.bfloat16)],
    )(o3, wo3, bo.reshape(1, D), hin, s2.reshape(1, D), b2.reshape(1, D))


def _ffn_kernel(flat_ref, m_ref, w1_ref, b1_ref, w2_ref, b2_ref, out_ref):
    e = pl.program_id(0)
    flat = flat_ref[...]
    slot = jax.lax.broadcasted_iota(jnp.int32, (S, CAP), 1).astype(
        jnp.float32) + (jnp.float32(CAP) * e.astype(jnp.float32))
    p = jnp.where(flat == slot, jnp.float32(1.0),
                  jnp.float32(0.0)).astype(jnp.bfloat16)
    ein = jax.lax.dot_general(p, m_ref[...], (((0,), (0,)), ((), ())),
                              preferred_element_type=jnp.float32)
    hid = jax.lax.dot(ein.astype(jnp.bfloat16), w1_ref[0],
                      preferred_element_type=jnp.float32)
    hid = jnp.maximum(hid + b1_ref[0], 0.0)
    out_ref[...] = (jax.lax.dot(hid.astype(jnp.bfloat16), w2_ref[0],
                                preferred_element_type=jnp.float32)
                    + b2_ref[0]).astype(jnp.bfloat16)


def _ffn(flat, m_in, w1, b1, w2, b2):
    return pl.pallas_call(
        _ffn_kernel,
        grid=(E,),
        in_specs=[pl.BlockSpec((S, 1), lambda e: (0, 0)),
                  pl.BlockSpec((S, D), lambda e: (0, 0)),
                  pl.BlockSpec((1, D, F), lambda e: (e, 0, 0)),
                  pl.BlockSpec((1, 1, F), lambda e: (e, 0, 0)),
                  pl.BlockSpec((1, F, D), lambda e: (e, 0, 0)),
                  pl.BlockSpec((1, 1, D), lambda e: (e, 0, 0))],
        out_specs=pl.BlockSpec((CAP, D), lambda e: (e, 0)),
        out_shape=jax.ShapeDtypeStruct((ECAP, D), jnp.bfloat16),
    )(flat, m_in, w1, b1.reshape(E, 1, F), w2, b2.reshape(E, 1, D))


_TB = 512


def _combine_kernel(flat_ref, h_ref, eout_ref, o_ref):
    flat = flat_ref[...]
    slot = jax.lax.broadcasted_iota(jnp.int32, (_TB, ECAP), 1).astype(
        jnp.float32)
    p = jnp.where(flat == slot, jnp.float32(1.0),
                  jnp.float32(0.0)).astype(jnp.bfloat16)
    y = jax.lax.dot(p, eout_ref[...], preferred_element_type=jnp.float32)
    o_ref[...] = h_ref[...] + y


def _combine(flat, h, eout):
    return pl.pallas_call(
        _combine_kernel,
        grid=(S // _TB,),
        in_specs=[pl.BlockSpec((_TB, 1), lambda t: (t, 0)),
                  pl.BlockSpec((_TB, D), lambda t: (t, 0)),
                  pl.BlockSpec((ECAP, D), lambda t: (0, 0))],
        out_specs=pl.BlockSpec((_TB, D), lambda t: (t, 0)),
        out_shape=jax.ShapeDtypeStruct((S, D), jnp.float32),
    )(flat, h, eout)


def kernel(x, Wqkv, bqkv, Wo, bo, ln1_s, ln1_b, ln2_s, ln2_b, W1, b1, W2, b2):
    xs = x[0]
    flat = _routing(xs[:, :128])

    h = xs
    for l in range(L):
        a = _layernorm(h, ln1_s[l], ln1_b[l])
        wqkv16 = Wqkv[l].astype(jnp.bfloat16)
        wq3 = wqkv16[:, :D].reshape(D, H, DH).transpose(1, 0, 2)
        wk3 = wqkv16[:, D:2 * D].reshape(D, H, DH).transpose(1, 0, 2)
        wv3 = wqkv16[:, 2 * D:].reshape(D, H, DH).transpose(1, 0, 2)
        bq3 = bqkv[l, :D].reshape(H, 1, DH)
        bk3 = bqkv[l, D:2 * D].reshape(H, 1, DH)
        bv3 = bqkv[l, 2 * D:].reshape(H, 1, DH)
        q3, k3, v3 = _qkv(a, wq3, wk3, wv3, bq3, bk3, bv3)
        o3 = _attention(q3, k3, v3)
        wo3 = Wo[l].astype(jnp.bfloat16).reshape(H, DH, D)
        h, m_in = _proj(o3, wo3, bo[l], h, ln2_s[l], ln2_b[l])
        eout = _ffn(flat, m_in, W1[l].astype(jnp.bfloat16), b1[l],
                    W2[l].astype(jnp.bfloat16), b2[l])
        h = _combine(flat, h, eout)

    return h.reshape(B, S, D)
